# Initial kernel scaffold; baseline (speedup 1.0000x reference)
import jax
import jax.numpy as jnp
from jax.experimental import pallas as pl


def kernel(inputs, hx, rows, cols, vals, W_ru, b_ru, W_c, b_c):
    raise NotImplementedError("write your pallas kernel here")


# speedup vs baseline: 2.1637x; 2.1637x over previous
import functools

import jax
import jax.numpy as jnp
from jax import lax
from jax.experimental import pallas as pl
from jax.experimental.pallas import tpu as pltpu
from jax.experimental.pallas import tpu_sc as plsc

N = 10000
NNZ = 320000
UNITS = 128
IN_DIM = 2
B = 4
F = IN_DIM + UNITS
FP = 144
M = B * N

NUM_TILES = 16
E = 80
EDGES_PER_TILE = NNZ // NUM_TILES
NCHUNK = EDGES_PER_TILE // E
NP = 10240
ROWS_PER_TILE = NP // NUM_TILES
LAST_ROWS = N - 15 * ROWS_PER_TILE


def _spmm2_body(table, rows, cols, vals, zeros, out1, out2,
                colv, rowv, valv, gbuf, slab, sem):
    core = lax.axis_index("c")
    sub = lax.axis_index("s")
    ebase = sub * EDGES_PER_TILE
    rbase = sub * ROWS_PER_TILE

    for hop in range(2):
        src = table if hop == 0 else out1
        dst = out1 if hop == 0 else out2
        for b_i in range(2):
            b = 2 * core + b_i
            boff = b * N
            pltpu.sync_copy(zeros.at[pl.ds(rbase, ROWS_PER_TILE)],
                            slab.at[pl.ds(rbase, ROWS_PER_TILE)])
            plsc.subcore_barrier()

            def chunk_body(i, _, boff=boff, src=src):
                off = ebase + i * E
                pltpu.sync_copy(cols.at[pl.ds(off, E)], colv)
                pltpu.sync_copy(rows.at[pl.ds(off, E)], rowv)
                pltpu.sync_copy(vals.at[pl.ds(off, E)], valv.at[pl.ds(0, E)])
                for j in range(E // 16):
                    colv[pl.ds(j * 16, 16)] = colv[pl.ds(j * 16, 16)] + boff
                pltpu.async_copy(src.at[colv], gbuf, sem).wait()

                def scale_body(e, _):
                    v16 = valv[pl.ds(e, 16)]
                    vv = jnp.full((16,), v16[0], dtype=jnp.float32)
                    for j in range(FP // 16):
                        gbuf[e, pl.ds(j * 16, 16)] = gbuf[e, pl.ds(j * 16, 16)] * vv
                    return 0

                lax.fori_loop(0, E, scale_body, 0, unroll=2)
                pltpu.sync_copy(gbuf, slab.at[rowv], add=True)
                return 0

            lax.fori_loop(0, NCHUNK, chunk_body, 0)
            plsc.subcore_barrier()

            @pl.when(sub < NUM_TILES - 1)
            def _():
                pltpu.sync_copy(slab.at[pl.ds(rbase, ROWS_PER_TILE)],
                                dst.at[pl.ds(boff + rbase, ROWS_PER_TILE)])

            @pl.when(sub == NUM_TILES - 1)
            def _():
                pltpu.sync_copy(slab.at[pl.ds(rbase, LAST_ROWS)],
                                dst.at[pl.ds(boff + rbase, LAST_ROWS)])
        plsc.subcore_barrier()


_spmm2 = functools.partial(
    pl.kernel,
    out_type=(jax.ShapeDtypeStruct((M, FP), jnp.float32),
              jax.ShapeDtypeStruct((M, FP), jnp.float32)),
    mesh=plsc.VectorSubcoreMesh(core_axis_name="c", subcore_axis_name="s"),
    scratch_types=[
        pltpu.VMEM((E,), jnp.int32),
        pltpu.VMEM((E,), jnp.int32),
        pltpu.VMEM((E + 16,), jnp.float32),
        pltpu.VMEM((E, FP), jnp.float32),
        pltpu.VMEM_SHARED((NP, FP), jnp.float32),
        pltpu.SemaphoreType.DMA,
    ],
    compiler_params=pltpu.CompilerParams(use_tc_tiling_on_sc=False),
)(_spmm2_body)


BM = 2000


def _ru_body(x0, y1, y2, w, bias, hx, u_out, rh_out):
    w0 = w[0] - w[2]
    w1 = w[1]
    w2 = 2.0 * w[2]
    acc = jnp.dot(x0[...], w0, preferred_element_type=jnp.float32)
    acc += jnp.dot(y1[...], w1, preferred_element_type=jnp.float32)
    acc += jnp.dot(y2[...], w2, preferred_element_type=jnp.float32)
    acc += bias[...]
    val = jax.nn.sigmoid(acc)
    u_out[...] = val[:, UNITS:]
    rh_out[...] = val[:, :UNITS] * hx[...]


def _c_body(x0, y1, y2, w, bias, u, hx, out):
    w0 = w[0] - w[2]
    w1 = w[1]
    w2 = 2.0 * w[2]
    acc = jnp.dot(x0[...], w0, preferred_element_type=jnp.float32)
    acc += jnp.dot(y1[...], w1, preferred_element_type=jnp.float32)
    acc += jnp.dot(y2[...], w2, preferred_element_type=jnp.float32)
    acc += bias[...]
    c = jnp.tanh(acc)
    uu = u[...]
    out[...] = uu * hx[...] + (1.0 - uu) * c


def _row_spec(width):
    return pl.BlockSpec((BM, width), lambda i: (i, 0))


def _ru(x0, y1, y2, w, bias, hx):
    return pl.pallas_call(
        _ru_body,
        grid=(M // BM,),
        in_specs=[
            _row_spec(FP), _row_spec(FP), _row_spec(FP),
            pl.BlockSpec((3, FP, 2 * UNITS), lambda i: (0, 0, 0)),
            pl.BlockSpec((1, 2 * UNITS), lambda i: (0, 0)),
            _row_spec(UNITS),
        ],
        out_specs=[_row_spec(UNITS), _row_spec(UNITS)],
        out_shape=[jax.ShapeDtypeStruct((M, UNITS), jnp.float32)] * 2,
    )(x0, y1, y2, w, bias, hx)


def _c(x0, y1, y2, w, bias, u, hx):
    return pl.pallas_call(
        _c_body,
        grid=(M // BM,),
        in_specs=[
            _row_spec(FP), _row_spec(FP), _row_spec(FP),
            pl.BlockSpec((3, FP, UNITS), lambda i: (0, 0, 0)),
            pl.BlockSpec((1, UNITS), lambda i: (0, 0)),
            _row_spec(UNITS), _row_spec(UNITS),
        ],
        out_specs=_row_spec(UNITS),
        out_shape=jax.ShapeDtypeStruct((M, UNITS), jnp.float32),
    )(x0, y1, y2, w, bias, u, hx)


def kernel(inputs, hx, rows, cols, vals, W_ru, b_ru, W_c, b_c):
    xi = inputs.reshape(B, N, IN_DIM)
    h = hx.reshape(B, N, UNITS)
    pad = jnp.zeros((B, N, FP - F), jnp.float32)
    tab1 = jnp.concatenate([xi, h, pad], axis=-1).reshape(M, FP)
    zeros = jnp.zeros((NP, FP), jnp.float32)
    wru = jnp.pad(W_ru.reshape(F, 3, 2 * UNITS).transpose(1, 0, 2),
                  ((0, 0), (0, FP - F), (0, 0)))
    wc = jnp.pad(W_c.reshape(F, 3, UNITS).transpose(1, 0, 2),
                 ((0, 0), (0, FP - F), (0, 0)))
    h2 = h.reshape(M, UNITS)

    y1, y2 = _spmm2(tab1, rows, cols, vals, zeros)
    u, rh = _ru(tab1, y1, y2, wru, b_ru.reshape(1, -1), h2)
    tab2 = jnp.concatenate(
        [xi.reshape(M, IN_DIM), rh, jnp.zeros((M, FP - F), jnp.float32)], axis=-1)
    y1c, y2c = _spmm2(tab2, rows, cols, vals, zeros)
    out = _c(tab2, y1c, y2c, wc, b_c.reshape(1, -1), u, h2)
    return out.reshape(B, N * UNITS)


# speedup vs baseline: 4.8535x; 2.2432x over previous
import functools

import jax
import jax.numpy as jnp
from jax import lax
from jax.experimental import pallas as pl
from jax.experimental.pallas import tpu as pltpu
from jax.experimental.pallas import tpu_sc as plsc

N = 10000
NNZ = 320000
UNITS = 128
IN_DIM = 2
B = 4
F = IN_DIM + UNITS
FP = 144
M = B * N

NUM_TILES = 16
E = 80
EDGES_PER_TILE = NNZ // NUM_TILES
NCHUNK = EDGES_PER_TILE // E
GCHUNK = 50
GEDGES = GCHUNK * E
NP = 10240
ROWS_PER_TILE = NP // NUM_TILES
LAST_ROWS = N - 15 * ROWS_PER_TILE


def _spmm2_body(table, rows2d, cols2d, vals, zeros, out1, out2,
                rowbuf, colbuf, valbuf, colv0, colv1, gbuf0, gbuf1, slab,
                gsem0, gsem1, ssem0, ssem1):
    core = lax.axis_index("c")
    sub = lax.axis_index("s")
    rbase = sub * ROWS_PER_TILE
    cbase = sub * NCHUNK

    def build(i, colv, boff):
        for j in range(E // 16):
            colv[pl.ds(j * 16, 16)] = colbuf[i, pl.ds(j * 16, 16)] + boff

    def scale(i, gbuf):
        base = i * E

        def scale_body(e, _):
            v16 = valbuf[pl.ds(base + e, 16)]
            vv = jnp.full((16,), v16[0], dtype=jnp.float32)
            for j in range(FP // 16):
                gbuf[e, pl.ds(j * 16, 16)] = gbuf[e, pl.ds(j * 16, 16)] * vv
            return 0

        lax.fori_loop(0, E, scale_body, 0, unroll=2)

    for hop in range(2):
        src = table if hop == 0 else out1
        dst = out1 if hop == 0 else out2
        for b_i in range(2):
            b = 2 * core + b_i
            boff = b * N
            pltpu.sync_copy(zeros.at[pl.ds(rbase, ROWS_PER_TILE)],
                            slab.at[pl.ds(rbase, ROWS_PER_TILE)])
            plsc.subcore_barrier()

            def group(grp, _, boff=boff, src=src):
                pltpu.sync_copy(rows2d.at[pl.ds(cbase + grp * GCHUNK, GCHUNK)],
                                rowbuf)
                pltpu.sync_copy(cols2d.at[pl.ds(cbase + grp * GCHUNK, GCHUNK)],
                                colbuf)
                pltpu.sync_copy(
                    vals.at[pl.ds(sub * EDGES_PER_TILE + grp * GEDGES, GEDGES)],
                    valbuf.at[pl.ds(0, GEDGES)])

                build(0, colv0, boff)
                pltpu.async_copy(src.at[colv0], gbuf0, gsem0)
                build(1, colv1, boff)
                pltpu.async_copy(src.at[colv1], gbuf1, gsem1)

                def pair(g, _):
                    i0 = 2 * g
                    i1 = i0 + 1
                    pltpu.make_async_copy(src.at[colv0], gbuf0, gsem0).wait()
                    scale(i0, gbuf0)
                    pltpu.async_copy(gbuf0, slab.at[rowbuf.at[i0]], ssem0, add=True)
                    pltpu.make_async_copy(src.at[colv1], gbuf1, gsem1).wait()
                    scale(i1, gbuf1)
                    pltpu.async_copy(gbuf1, slab.at[rowbuf.at[i1]], ssem1, add=True)
                    pltpu.make_async_copy(gbuf0, slab.at[rowbuf.at[i0]], ssem0).wait()
                    build(i0 + 2, colv0, boff)
                    pltpu.async_copy(src.at[colv0], gbuf0, gsem0)
                    pltpu.make_async_copy(gbuf1, slab.at[rowbuf.at[i1]], ssem1).wait()
                    build(i1 + 2, colv1, boff)
                    pltpu.async_copy(src.at[colv1], gbuf1, gsem1)
                    return 0

                lax.fori_loop(0, GCHUNK // 2 - 1, pair, 0)

                i0 = GCHUNK - 2
                i1 = GCHUNK - 1
                pltpu.make_async_copy(src.at[colv0], gbuf0, gsem0).wait()
                scale(i0, gbuf0)
                pltpu.async_copy(gbuf0, slab.at[rowbuf.at[i0]], ssem0, add=True)
                pltpu.make_async_copy(src.at[colv1], gbuf1, gsem1).wait()
                scale(i1, gbuf1)
                pltpu.async_copy(gbuf1, slab.at[rowbuf.at[i1]], ssem1, add=True)
                pltpu.make_async_copy(gbuf0, slab.at[rowbuf.at[i0]], ssem0).wait()
                pltpu.make_async_copy(gbuf1, slab.at[rowbuf.at[i1]], ssem1).wait()
                return 0

            lax.fori_loop(0, NCHUNK // GCHUNK, group, 0)
            plsc.subcore_barrier()

            @pl.when(sub < NUM_TILES - 1)
            def _():
                pltpu.sync_copy(slab.at[pl.ds(rbase, ROWS_PER_TILE)],
                                dst.at[pl.ds(boff + rbase, ROWS_PER_TILE)])

            @pl.when(sub == NUM_TILES - 1)
            def _():
                pltpu.sync_copy(slab.at[pl.ds(rbase, LAST_ROWS)],
                                dst.at[pl.ds(boff + rbase, LAST_ROWS)])
        plsc.subcore_barrier()


_spmm2 = functools.partial(
    pl.kernel,
    out_type=(jax.ShapeDtypeStruct((M, FP), jnp.float32),
              jax.ShapeDtypeStruct((M, FP), jnp.float32)),
    mesh=plsc.VectorSubcoreMesh(core_axis_name="c", subcore_axis_name="s"),
    scratch_types=[
        pltpu.VMEM((GCHUNK, E), jnp.int32),
        pltpu.VMEM((GCHUNK, E), jnp.int32),
        pltpu.VMEM((GEDGES + 16,), jnp.float32),
        pltpu.VMEM((E,), jnp.int32),
        pltpu.VMEM((E,), jnp.int32),
        pltpu.VMEM((E, FP), jnp.float32),
        pltpu.VMEM((E, FP), jnp.float32),
        pltpu.VMEM_SHARED((NP, FP), jnp.float32),
        pltpu.SemaphoreType.DMA,
        pltpu.SemaphoreType.DMA,
        pltpu.SemaphoreType.DMA,
        pltpu.SemaphoreType.DMA,
    ],
    compiler_params=pltpu.CompilerParams(use_tc_tiling_on_sc=False),
)(_spmm2_body)


BM = 2000


def _ru_body(x0, y1, y2, w, bias, hx, u_out, rh_out):
    w0 = w[0] - w[2]
    w1 = w[1]
    w2 = 2.0 * w[2]
    acc = jnp.dot(x0[...], w0, preferred_element_type=jnp.float32)
    acc += jnp.dot(y1[...], w1, preferred_element_type=jnp.float32)
    acc += jnp.dot(y2[...], w2, preferred_element_type=jnp.float32)
    acc += bias[...]
    val = jax.nn.sigmoid(acc)
    u_out[...] = val[:, UNITS:]
    rh_out[...] = val[:, :UNITS] * hx[...]


def _c_body(x0, y1, y2, w, bias, u, hx, out):
    w0 = w[0] - w[2]
    w1 = w[1]
    w2 = 2.0 * w[2]
    acc = jnp.dot(x0[...], w0, preferred_element_type=jnp.float32)
    acc += jnp.dot(y1[...], w1, preferred_element_type=jnp.float32)
    acc += jnp.dot(y2[...], w2, preferred_element_type=jnp.float32)
    acc += bias[...]
    c = jnp.tanh(acc)
    uu = u[...]
    out[...] = uu * hx[...] + (1.0 - uu) * c


def _row_spec(width):
    return pl.BlockSpec((BM, width), lambda i: (i, 0))


def _ru(x0, y1, y2, w, bias, hx):
    return pl.pallas_call(
        _ru_body,
        grid=(M // BM,),
        in_specs=[
            _row_spec(FP), _row_spec(FP), _row_spec(FP),
            pl.BlockSpec((3, FP, 2 * UNITS), lambda i: (0, 0, 0)),
            pl.BlockSpec((1, 2 * UNITS), lambda i: (0, 0)),
            _row_spec(UNITS),
        ],
        out_specs=[_row_spec(UNITS), _row_spec(UNITS)],
        out_shape=[jax.ShapeDtypeStruct((M, UNITS), jnp.float32)] * 2,
    )(x0, y1, y2, w, bias, hx)


def _c(x0, y1, y2, w, bias, u, hx):
    return pl.pallas_call(
        _c_body,
        grid=(M // BM,),
        in_specs=[
            _row_spec(FP), _row_spec(FP), _row_spec(FP),
            pl.BlockSpec((3, FP, UNITS), lambda i: (0, 0, 0)),
            pl.BlockSpec((1, UNITS), lambda i: (0, 0)),
            _row_spec(UNITS), _row_spec(UNITS),
        ],
        out_specs=_row_spec(UNITS),
        out_shape=jax.ShapeDtypeStruct((M, UNITS), jnp.float32),
    )(x0, y1, y2, w, bias, u, hx)


def kernel(inputs, hx, rows, cols, vals, W_ru, b_ru, W_c, b_c):
    xi = inputs.reshape(B, N, IN_DIM)
    h = hx.reshape(B, N, UNITS)
    pad = jnp.zeros((B, N, FP - F), jnp.float32)
    tab1 = jnp.concatenate([xi, h, pad], axis=-1).reshape(M, FP)
    zeros = jnp.zeros((NP, FP), jnp.float32)
    wru = jnp.pad(W_ru.reshape(F, 3, 2 * UNITS).transpose(1, 0, 2),
                  ((0, 0), (0, FP - F), (0, 0)))
    wc = jnp.pad(W_c.reshape(F, 3, UNITS).transpose(1, 0, 2),
                 ((0, 0), (0, FP - F), (0, 0)))
    h2 = h.reshape(M, UNITS)

    rows2d = rows.reshape(NNZ // E, E)
    cols2d = cols.reshape(NNZ // E, E)
    y1, y2 = _spmm2(tab1, rows2d, cols2d, vals, zeros)
    u, rh = _ru(tab1, y1, y2, wru, b_ru.reshape(1, -1), h2)
    tab2 = jnp.concatenate(
        [xi.reshape(M, IN_DIM), rh, jnp.zeros((M, FP - F), jnp.float32)], axis=-1)
    y1c, y2c = _spmm2(tab2, rows2d, cols2d, vals, zeros)
    out = _c(tab2, y1c, y2c, wc, b_c.reshape(1, -1), u, h2)
    return out.reshape(B, N * UNITS)


# speedup vs baseline: 5.3563x; 1.1036x over previous
import functools

import jax
import jax.numpy as jnp
from jax import lax
from jax.experimental import pallas as pl
from jax.experimental.pallas import tpu as pltpu
from jax.experimental.pallas import tpu_sc as plsc

N = 10000
NNZ = 320000
UNITS = 128
IN_DIM = 2
B = 4
F = IN_DIM + UNITS
FP = 144
M = B * N

NUM_TILES = 16
E = 80
EDGES_PER_TILE = NNZ // NUM_TILES
NCHUNK = EDGES_PER_TILE // E
GCHUNK = 50
GEDGES = GCHUNK * E
NP = 10240
ROWS_PER_TILE = NP // NUM_TILES
LAST_ROWS = N - 15 * ROWS_PER_TILE


def _spmm2_body(table, rows2d, cols2d, vals, zeros, out1, out2,
                rowbuf, colbuf, valbuf, colv0, colv1, gbuf0, gbuf1, slab,
                gsem0, gsem1, ssem0, ssem1):
    core = lax.axis_index("c")
    sub = lax.axis_index("s")
    rbase = sub * ROWS_PER_TILE
    cbase = sub * NCHUNK

    def build(i, colv, boff):
        for j in range(E // 16):
            colv[pl.ds(j * 16, 16)] = colbuf[i, pl.ds(j * 16, 16)] + boff

    def scale(i, gbuf):
        base = i * E

        @plsc.parallel_loop(0, E, unroll=4)
        def _(e):
            v16 = valbuf[pl.ds(base + e, 16)]
            vv = jnp.full((16,), v16[0], dtype=jnp.float32)
            for j in range(FP // 16):
                gbuf[e, pl.ds(j * 16, 16)] = gbuf[e, pl.ds(j * 16, 16)] * vv

    for hop in range(2):
        src = table if hop == 0 else out1
        dst = out1 if hop == 0 else out2
        for b_i in range(2):
            b = 2 * core + b_i
            boff = b * N
            pltpu.sync_copy(zeros.at[pl.ds(rbase, ROWS_PER_TILE)],
                            slab.at[pl.ds(rbase, ROWS_PER_TILE)])
            plsc.subcore_barrier()

            def group(grp, _, boff=boff, src=src):
                pltpu.sync_copy(rows2d.at[pl.ds(cbase + grp * GCHUNK, GCHUNK)],
                                rowbuf)
                pltpu.sync_copy(cols2d.at[pl.ds(cbase + grp * GCHUNK, GCHUNK)],
                                colbuf)
                pltpu.sync_copy(
                    vals.at[pl.ds(sub * EDGES_PER_TILE + grp * GEDGES, GEDGES)],
                    valbuf.at[pl.ds(0, GEDGES)])

                build(0, colv0, boff)
                pltpu.async_copy(src.at[colv0], gbuf0, gsem0)
                build(1, colv1, boff)
                pltpu.async_copy(src.at[colv1], gbuf1, gsem1)

                def pair(g, _):
                    i0 = 2 * g
                    i1 = i0 + 1
                    pltpu.make_async_copy(src.at[colv0], gbuf0, gsem0).wait()
                    scale(i0, gbuf0)
                    pltpu.async_copy(gbuf0, slab.at[rowbuf.at[i0]], ssem0, add=True)
                    pltpu.make_async_copy(src.at[colv1], gbuf1, gsem1).wait()
                    scale(i1, gbuf1)
                    pltpu.async_copy(gbuf1, slab.at[rowbuf.at[i1]], ssem1, add=True)
                    pltpu.make_async_copy(gbuf0, slab.at[rowbuf.at[i0]], ssem0).wait()
                    build(i0 + 2, colv0, boff)
                    pltpu.async_copy(src.at[colv0], gbuf0, gsem0)
                    pltpu.make_async_copy(gbuf1, slab.at[rowbuf.at[i1]], ssem1).wait()
                    build(i1 + 2, colv1, boff)
                    pltpu.async_copy(src.at[colv1], gbuf1, gsem1)
                    return 0

                lax.fori_loop(0, GCHUNK // 2 - 1, pair, 0)

                i0 = GCHUNK - 2
                i1 = GCHUNK - 1
                pltpu.make_async_copy(src.at[colv0], gbuf0, gsem0).wait()
                scale(i0, gbuf0)
                pltpu.async_copy(gbuf0, slab.at[rowbuf.at[i0]], ssem0, add=True)
                pltpu.make_async_copy(src.at[colv1], gbuf1, gsem1).wait()
                scale(i1, gbuf1)
                pltpu.async_copy(gbuf1, slab.at[rowbuf.at[i1]], ssem1, add=True)
                pltpu.make_async_copy(gbuf0, slab.at[rowbuf.at[i0]], ssem0).wait()
                pltpu.make_async_copy(gbuf1, slab.at[rowbuf.at[i1]], ssem1).wait()
                return 0

            lax.fori_loop(0, NCHUNK // GCHUNK, group, 0)
            plsc.subcore_barrier()

            @pl.when(sub < NUM_TILES - 1)
            def _():
                pltpu.sync_copy(slab.at[pl.ds(rbase, ROWS_PER_TILE)],
                                dst.at[pl.ds(boff + rbase, ROWS_PER_TILE)])

            @pl.when(sub == NUM_TILES - 1)
            def _():
                pltpu.sync_copy(slab.at[pl.ds(rbase, LAST_ROWS)],
                                dst.at[pl.ds(boff + rbase, LAST_ROWS)])
        plsc.subcore_barrier()


_spmm2 = functools.partial(
    pl.kernel,
    out_type=(jax.ShapeDtypeStruct((M, FP), jnp.float32),
              jax.ShapeDtypeStruct((M, FP), jnp.float32)),
    mesh=plsc.VectorSubcoreMesh(core_axis_name="c", subcore_axis_name="s"),
    scratch_types=[
        pltpu.VMEM((GCHUNK, E), jnp.int32),
        pltpu.VMEM((GCHUNK, E), jnp.int32),
        pltpu.VMEM((GEDGES + 16,), jnp.float32),
        pltpu.VMEM((E,), jnp.int32),
        pltpu.VMEM((E,), jnp.int32),
        pltpu.VMEM((E, FP), jnp.float32),
        pltpu.VMEM((E, FP), jnp.float32),
        pltpu.VMEM_SHARED((NP, FP), jnp.float32),
        pltpu.SemaphoreType.DMA,
        pltpu.SemaphoreType.DMA,
        pltpu.SemaphoreType.DMA,
        pltpu.SemaphoreType.DMA,
    ],
    compiler_params=pltpu.CompilerParams(use_tc_tiling_on_sc=False),
)(_spmm2_body)


BM = 2000


def _ru_body(x0, y1, y2, w, bias, hx, u_out, rh_out):
    w0 = w[0] - w[2]
    w1 = w[1]
    w2 = 2.0 * w[2]
    acc = jnp.dot(x0[...], w0, preferred_element_type=jnp.float32)
    acc += jnp.dot(y1[...], w1, preferred_element_type=jnp.float32)
    acc += jnp.dot(y2[...], w2, preferred_element_type=jnp.float32)
    acc += bias[...]
    val = jax.nn.sigmoid(acc)
    u_out[...] = val[:, UNITS:]
    rh_out[...] = val[:, :UNITS] * hx[...]


def _c_body(x0, y1, y2, w, bias, u, hx, out):
    w0 = w[0] - w[2]
    w1 = w[1]
    w2 = 2.0 * w[2]
    acc = jnp.dot(x0[...], w0, preferred_element_type=jnp.float32)
    acc += jnp.dot(y1[...], w1, preferred_element_type=jnp.float32)
    acc += jnp.dot(y2[...], w2, preferred_element_type=jnp.float32)
    acc += bias[...]
    c = jnp.tanh(acc)
    uu = u[...]
    out[...] = uu * hx[...] + (1.0 - uu) * c


def _row_spec(width):
    return pl.BlockSpec((BM, width), lambda i: (i, 0))


def _ru(x0, y1, y2, w, bias, hx):
    return pl.pallas_call(
        _ru_body,
        grid=(M // BM,),
        in_specs=[
            _row_spec(FP), _row_spec(FP), _row_spec(FP),
            pl.BlockSpec((3, FP, 2 * UNITS), lambda i: (0, 0, 0)),
            pl.BlockSpec((1, 2 * UNITS), lambda i: (0, 0)),
            _row_spec(UNITS),
        ],
        out_specs=[_row_spec(UNITS), _row_spec(UNITS)],
        out_shape=[jax.ShapeDtypeStruct((M, UNITS), jnp.float32)] * 2,
    )(x0, y1, y2, w, bias, hx)


def _c(x0, y1, y2, w, bias, u, hx):
    return pl.pallas_call(
        _c_body,
        grid=(M // BM,),
        in_specs=[
            _row_spec(FP), _row_spec(FP), _row_spec(FP),
            pl.BlockSpec((3, FP, UNITS), lambda i: (0, 0, 0)),
            pl.BlockSpec((1, UNITS), lambda i: (0, 0)),
            _row_spec(UNITS), _row_spec(UNITS),
        ],
        out_specs=_row_spec(UNITS),
        out_shape=jax.ShapeDtypeStruct((M, UNITS), jnp.float32),
    )(x0, y1, y2, w, bias, u, hx)


def kernel(inputs, hx, rows, cols, vals, W_ru, b_ru, W_c, b_c):
    xi = inputs.reshape(B, N, IN_DIM)
    h = hx.reshape(B, N, UNITS)
    pad = jnp.zeros((B, N, FP - F), jnp.float32)
    tab1 = jnp.concatenate([xi, h, pad], axis=-1).reshape(M, FP)
    zeros = jnp.zeros((NP, FP), jnp.float32)
    wru = jnp.pad(W_ru.reshape(F, 3, 2 * UNITS).transpose(1, 0, 2),
                  ((0, 0), (0, FP - F), (0, 0)))
    wc = jnp.pad(W_c.reshape(F, 3, UNITS).transpose(1, 0, 2),
                 ((0, 0), (0, FP - F), (0, 0)))
    h2 = h.reshape(M, UNITS)

    rows2d = rows.reshape(NNZ // E, E)
    cols2d = cols.reshape(NNZ // E, E)
    y1, y2 = _spmm2(tab1, rows2d, cols2d, vals, zeros)
    u, rh = _ru(tab1, y1, y2, wru, b_ru.reshape(1, -1), h2)
    tab2 = jnp.concatenate(
        [xi.reshape(M, IN_DIM), rh, jnp.zeros((M, FP - F), jnp.float32)], axis=-1)
    y1c, y2c = _spmm2(tab2, rows2d, cols2d, vals, zeros)
    out = _c(tab2, y1c, y2c, wc, b_c.reshape(1, -1), u, h2)
    return out.reshape(B, N * UNITS)


# speedup vs baseline: 6.2593x; 1.1686x over previous
import functools

import jax
import jax.numpy as jnp
from jax import lax
from jax.experimental import pallas as pl
from jax.experimental.pallas import tpu as pltpu
from jax.experimental.pallas import tpu_sc as plsc

N = 10000
NNZ = 320000
UNITS = 128
IN_DIM = 2
B = 4
M = B * N
XW = 16

NUM_TILES = 16
E = 80
EDGES_PER_TILE = NNZ // NUM_TILES
NCHUNK = EDGES_PER_TILE // E
GCHUNK = 50
GEDGES = GCHUNK * E
NP = 10240
ROWS_PER_TILE = NP // NUM_TILES
LAST_ROWS = N - 15 * ROWS_PER_TILE


def _mk_sc_pipeline(width, dtype):
    def run(src, rows2d, cols2d, vals, slab, rowbuf, colbuf, valbuf,
            colv0, colv1, g0, g1, gsem0, gsem1, ssem0, ssem1, sub, boff):
        cbase = sub * NCHUNK

        def build(i, colv):
            for j in range(E // 16):
                colv[pl.ds(j * 16, 16)] = colbuf[i, pl.ds(j * 16, 16)] + boff

        def scale(i, gbuf):
            base = i * E

            @plsc.parallel_loop(0, E, unroll=4)
            def _(e):
                if dtype == jnp.bfloat16:
                    vv = valbuf[base + e, pl.ds(0, 32)]
                    for j in range(width // 32):
                        gbuf[e, pl.ds(j * 32, 32)] = (
                            gbuf[e, pl.ds(j * 32, 32)] * vv)
                else:
                    v16 = valbuf[pl.ds(base + e, 16)]
                    vv = jnp.full((16,), v16[0], dtype=jnp.float32)
                    for j in range(width // 16):
                        gbuf[e, pl.ds(j * 16, 16)] = (
                            gbuf[e, pl.ds(j * 16, 16)] * vv)

        def group(grp, _):
            pltpu.sync_copy(rows2d.at[pl.ds(cbase + grp * GCHUNK, GCHUNK)],
                            rowbuf)
            pltpu.sync_copy(cols2d.at[pl.ds(cbase + grp * GCHUNK, GCHUNK)],
                            colbuf)
            if dtype == jnp.bfloat16:
                pltpu.sync_copy(
                    vals.at[pl.ds(sub * EDGES_PER_TILE + grp * GEDGES, GEDGES)],
                    valbuf)
            else:
                pltpu.sync_copy(
                    vals.at[pl.ds(sub * EDGES_PER_TILE + grp * GEDGES, GEDGES)],
                    valbuf.at[pl.ds(0, GEDGES)])

            build(0, colv0)
            pltpu.async_copy(src.at[colv0], g0, gsem0)
            build(1, colv1)
            pltpu.async_copy(src.at[colv1], g1, gsem1)

            def pair(g, _):
                i0 = 2 * g
                i1 = i0 + 1
                pltpu.make_async_copy(src.at[colv0], g0, gsem0).wait()
                scale(i0, g0)
                pltpu.async_copy(g0, slab.at[rowbuf.at[i0]], ssem0, add=True)
                pltpu.make_async_copy(src.at[colv1], g1, gsem1).wait()
                scale(i1, g1)
                pltpu.async_copy(g1, slab.at[rowbuf.at[i1]], ssem1, add=True)
                pltpu.make_async_copy(g0, slab.at[rowbuf.at[i0]], ssem0).wait()
                build(i0 + 2, colv0)
                pltpu.async_copy(src.at[colv0], g0, gsem0)
                pltpu.make_async_copy(g1, slab.at[rowbuf.at[i1]], ssem1).wait()
                build(i1 + 2, colv1)
                pltpu.async_copy(src.at[colv1], g1, gsem1)
                return 0

            lax.fori_loop(0, GCHUNK // 2 - 1, pair, 0)

            i0 = GCHUNK - 2
            i1 = GCHUNK - 1
            pltpu.make_async_copy(src.at[colv0], g0, gsem0).wait()
            scale(i0, g0)
            pltpu.async_copy(g0, slab.at[rowbuf.at[i0]], ssem0, add=True)
            pltpu.make_async_copy(src.at[colv1], g1, gsem1).wait()
            scale(i1, g1)
            pltpu.async_copy(g1, slab.at[rowbuf.at[i1]], ssem1, add=True)
            pltpu.make_async_copy(g0, slab.at[rowbuf.at[i0]], ssem0).wait()
            pltpu.make_async_copy(g1, slab.at[rowbuf.at[i1]], ssem1).wait()
            return 0

        lax.fori_loop(0, NCHUNK // GCHUNK, group, 0)

    return run


_run_bf = _mk_sc_pipeline(UNITS, jnp.bfloat16)
_run_xi = _mk_sc_pipeline(XW, jnp.float32)


def _writeback(slab, dst, sub, boff, width):
    rbase = sub * ROWS_PER_TILE

    @pl.when(sub < NUM_TILES - 1)
    def _():
        pltpu.sync_copy(slab.at[pl.ds(rbase, ROWS_PER_TILE)],
                        dst.at[pl.ds(boff + rbase, ROWS_PER_TILE)])

    @pl.when(sub == NUM_TILES - 1)
    def _():
        pltpu.sync_copy(slab.at[pl.ds(rbase, LAST_ROWS)],
                        dst.at[pl.ds(boff + rbase, LAST_ROWS)])


def _spmm2_bf_body(table, rows2d, cols2d, vals, zeros, out1, out2,
                   rowbuf, colbuf, valbuf, colv0, colv1, g0, g1, slab,
                   gsem0, gsem1, ssem0, ssem1):
    core = lax.axis_index("c")
    sub = lax.axis_index("s")
    rbase = sub * ROWS_PER_TILE

    for hop in range(2):
        src = table if hop == 0 else out1
        dst = out1 if hop == 0 else out2
        for b_i in range(2):
            b = 2 * core + b_i
            boff = b * N
            pltpu.sync_copy(zeros.at[pl.ds(rbase, ROWS_PER_TILE)],
                            slab.at[pl.ds(rbase, ROWS_PER_TILE)])
            plsc.subcore_barrier()
            _run_bf(src, rows2d, cols2d, vals, slab, rowbuf, colbuf, valbuf,
                    colv0, colv1, g0, g1, gsem0, gsem1, ssem0, ssem1, sub,
                    boff)
            plsc.subcore_barrier()
            _writeback(slab, dst, sub, boff, UNITS)
        plsc.subcore_barrier()


_spmm2_bf = functools.partial(
    pl.kernel,
    out_type=(jax.ShapeDtypeStruct((M, UNITS), jnp.bfloat16),
              jax.ShapeDtypeStruct((M, UNITS), jnp.bfloat16)),
    mesh=plsc.VectorSubcoreMesh(core_axis_name="c", subcore_axis_name="s"),
    scratch_types=[
        pltpu.VMEM((GCHUNK, E), jnp.int32),
        pltpu.VMEM((GCHUNK, E), jnp.int32),
        pltpu.VMEM((GEDGES, 32), jnp.bfloat16),
        pltpu.VMEM((E,), jnp.int32),
        pltpu.VMEM((E,), jnp.int32),
        pltpu.VMEM((E, UNITS), jnp.bfloat16),
        pltpu.VMEM((E, UNITS), jnp.bfloat16),
        pltpu.VMEM_SHARED((NP, UNITS), jnp.bfloat16),
        pltpu.SemaphoreType.DMA,
        pltpu.SemaphoreType.DMA,
        pltpu.SemaphoreType.DMA,
        pltpu.SemaphoreType.DMA,
    ],
    compiler_params=pltpu.CompilerParams(use_tc_tiling_on_sc=False),
)(_spmm2_bf_body)


def _xi_spmm_body(xisrc, rows2d, cols2d, vals, zeros, out1, out2,
                  rowbuf, colbuf, valbuf, colv0, colv1, g0, g1, slab,
                  gsem0, gsem1, ssem0, ssem1):
    core = lax.axis_index("c")
    sub = lax.axis_index("s")
    rbase = sub * ROWS_PER_TILE

    @pl.when(core == 0)
    def _():
        for hop in range(2):
            src = xisrc if hop == 0 else out1
            dst = out1 if hop == 0 else out2
            pltpu.sync_copy(zeros.at[pl.ds(rbase, ROWS_PER_TILE)],
                            slab.at[pl.ds(rbase, ROWS_PER_TILE)])
            plsc.subcore_barrier()
            _run_xi(src, rows2d, cols2d, vals, slab, rowbuf, colbuf, valbuf,
                    colv0, colv1, g0, g1, gsem0, gsem1, ssem0, ssem1, sub, 0)
            plsc.subcore_barrier()
            _writeback(slab, dst, sub, 0, XW)


_xi_spmm = functools.partial(
    pl.kernel,
    out_type=(jax.ShapeDtypeStruct((N, XW), jnp.float32),
              jax.ShapeDtypeStruct((N, XW), jnp.float32)),
    mesh=plsc.VectorSubcoreMesh(core_axis_name="c", subcore_axis_name="s"),
    scratch_types=[
        pltpu.VMEM((GCHUNK, E), jnp.int32),
        pltpu.VMEM((GCHUNK, E), jnp.int32),
        pltpu.VMEM((GEDGES + 16,), jnp.float32),
        pltpu.VMEM((E,), jnp.int32),
        pltpu.VMEM((E,), jnp.int32),
        pltpu.VMEM((E, XW), jnp.float32),
        pltpu.VMEM((E, XW), jnp.float32),
        pltpu.VMEM_SHARED((NP, XW), jnp.float32),
        pltpu.SemaphoreType.DMA,
        pltpu.SemaphoreType.DMA,
        pltpu.SemaphoreType.DMA,
        pltpu.SemaphoreType.DMA,
    ],
    compiler_params=pltpu.CompilerParams(use_tc_tiling_on_sc=False),
)(_xi_spmm_body)


BM = 2000
XK = 3 * IN_DIM


def _fold_w(wxi, wst):
    a0 = wxi[:, 0]
    a1 = wxi[:, 1]
    a2 = wxi[:, 2]
    xw = jnp.concatenate([a0 - a2, a1, 2.0 * a2], axis=0)
    return xw, wst[0] - wst[2], wst[1], 2.0 * wst[2]


def _ru_body(xim, h2, y1, y2, wxi, wst, bias, u_out, rhf_out, rhb_out):
    xw, w0, w1, w2 = _fold_w(wxi[...], wst)
    acc = jnp.dot(xim[...], xw, preferred_element_type=jnp.float32)
    acc += jnp.dot(h2[...], w0, preferred_element_type=jnp.float32)
    acc += jnp.dot(y1[...].astype(jnp.float32), w1,
                   preferred_element_type=jnp.float32)
    acc += jnp.dot(y2[...].astype(jnp.float32), w2,
                   preferred_element_type=jnp.float32)
    acc += bias[...]
    val = jax.nn.sigmoid(acc)
    u_out[...] = val[:, UNITS:]
    rhf = val[:, :UNITS] * h2[...]
    rhf_out[...] = rhf
    rhb_out[...] = rhf.astype(jnp.bfloat16)


def _c_body(xim, rhf, y1, y2, wxi, wst, bias, u, h2, out):
    xw, w0, w1, w2 = _fold_w(wxi[...], wst)
    acc = jnp.dot(xim[...], xw, preferred_element_type=jnp.float32)
    acc += jnp.dot(rhf[...], w0, preferred_element_type=jnp.float32)
    acc += jnp.dot(y1[...].astype(jnp.float32), w1,
                   preferred_element_type=jnp.float32)
    acc += jnp.dot(y2[...].astype(jnp.float32), w2,
                   preferred_element_type=jnp.float32)
    acc += bias[...]
    c = jnp.tanh(acc)
    uu = u[...]
    out[...] = uu * h2[...] + (1.0 - uu) * c


def _row_spec(width):
    return pl.BlockSpec((BM, width), lambda i: (i, 0))


def _full3(shape):
    return pl.BlockSpec(shape, lambda i: (0, 0, 0))


def _ru(xim, h2, y1, y2, wxi, wst, bias):
    ou = 2 * UNITS
    return pl.pallas_call(
        _ru_body,
        grid=(M // BM,),
        in_specs=[
            _row_spec(XK), _row_spec(UNITS), _row_spec(UNITS), _row_spec(UNITS),
            _full3((IN_DIM, 3, ou)), _full3((3, UNITS, ou)),
            pl.BlockSpec((1, ou), lambda i: (0, 0)),
        ],
        out_specs=[_row_spec(UNITS)] * 3,
        out_shape=[jax.ShapeDtypeStruct((M, UNITS), jnp.float32),
                   jax.ShapeDtypeStruct((M, UNITS), jnp.float32),
                   jax.ShapeDtypeStruct((M, UNITS), jnp.bfloat16)],
    )(xim, h2, y1, y2, wxi, wst, bias)


def _c(xim, rhf, y1, y2, wxi, wst, bias, u, h2):
    return pl.pallas_call(
        _c_body,
        grid=(M // BM,),
        in_specs=[
            _row_spec(XK), _row_spec(UNITS), _row_spec(UNITS), _row_spec(UNITS),
            _full3((IN_DIM, 3, UNITS)), _full3((3, UNITS, UNITS)),
            pl.BlockSpec((1, UNITS), lambda i: (0, 0)),
            _row_spec(UNITS), _row_spec(UNITS),
        ],
        out_specs=_row_spec(UNITS),
        out_shape=jax.ShapeDtypeStruct((M, UNITS), jnp.float32),
    )(xim, rhf, y1, y2, wxi, wst, bias, u, h2)


def _mk_xicols(a):
    return a[:, :B * IN_DIM].reshape(N, B, IN_DIM).transpose(1, 0, 2).reshape(
        M, IN_DIM)


def kernel(inputs, hx, rows, cols, vals, W_ru, b_ru, W_c, b_c):
    xi = inputs.reshape(B, N, IN_DIM)
    h2 = hx.reshape(M, UNITS)
    rows2d = rows.reshape(NNZ // E, E)
    cols2d = cols.reshape(NNZ // E, E)

    xi_all = jnp.zeros((N, XW), jnp.float32)
    xi_all = xi_all.at[:, :B * IN_DIM].set(
        xi.transpose(1, 0, 2).reshape(N, B * IN_DIM))
    zeros_xi = jnp.zeros((NP, XW), jnp.float32)
    zeros_bf = jnp.zeros((NP, UNITS), jnp.bfloat16)

    yxi1, yxi2 = _xi_spmm(xi_all, rows2d, cols2d, vals, zeros_xi)
    xim = jnp.concatenate(
        [xi.reshape(M, IN_DIM), _mk_xicols(yxi1), _mk_xicols(yxi2)], axis=-1)

    wk_ru = W_ru.reshape(IN_DIM + UNITS, 3, 2 * UNITS)
    wk_c = W_c.reshape(IN_DIM + UNITS, 3, UNITS)

    t1 = h2.astype(jnp.bfloat16)
    vals_bf = jnp.broadcast_to(vals.astype(jnp.bfloat16)[:, None], (NNZ, 32))
    y1, y2 = _spmm2_bf(t1, rows2d, cols2d, vals_bf, zeros_bf)
    u, rhf, rhb = _ru(xim, h2, y1, y2, wk_ru[:IN_DIM],
                      wk_ru[IN_DIM:].transpose(1, 0, 2), b_ru.reshape(1, -1))
    y1c, y2c = _spmm2_bf(rhb, rows2d, cols2d, vals_bf, zeros_bf)
    out = _c(xim, rhf, y1c, y2c, wk_c[:IN_DIM],
             wk_c[IN_DIM:].transpose(1, 0, 2), b_c.reshape(1, -1), u, h2)
    return out.reshape(B, N * UNITS)
